# baseline (speedup 1.0000x reference)
import functools

import jax
import jax.numpy as jnp
from jax import lax
from jax.experimental import pallas as pl
from jax.experimental.pallas import tpu as pltpu
from jax.experimental.pallas import tpu_sc as plsc

N = 10000
E = 320000
H = 128
G = 128
C = 10

NC = 2
NS = 16
NW = NC * NS
CH = 128
KCH = 80
PTE = KCH * CH
EPAD = NW * PTE
TOT = EPAD // CH
KAG = TOT // NS
NPAD = 10240
RPS = NPAD // NS

@functools.cache
def _sc_kernels():
    mesh = plsc.VectorSubcoreMesh(core_axis_name="c", subcore_axis_name="s",
                                  num_cores=NC, num_subcores=NS)

    @functools.partial(
        pl.kernel,
        out_type=jax.ShapeDtypeStruct((NC, NPAD), jnp.float32),
        mesh=mesh,
        scratch_types=[
            pltpu.VMEM_SHARED((NPAD,), jnp.float32),
            pltpu.VMEM((KCH, CH), jnp.int32),
            pltpu.VMEM((CH,), jnp.float32),
        ],
    )
    def sc_degree(dst3_hbm, zeros1_hbm, out_hbm, deg_sh, idst_v, ones_v):
        c = lax.axis_index("c")
        s = lax.axis_index("s")
        w = s * NC + c
        pltpu.sync_copy(zeros1_hbm.at[pl.ds(s * RPS, RPS)],
                        deg_sh.at[pl.ds(s * RPS, RPS)])
        for i in range(CH // 16):
            ones_v[pl.ds(i * 16, 16)] = jnp.full((16,), 1.0, jnp.float32)
        plsc.subcore_barrier()
        pltpu.sync_copy(dst3_hbm.at[w], idst_v)

        def body(j, carry):
            pltpu.sync_copy(ones_v, deg_sh.at[idst_v.at[j]], add=True)
            return carry

        lax.fori_loop(0, KCH, body, 0)
        plsc.subcore_barrier()
        pltpu.sync_copy(deg_sh.at[pl.ds(s * RPS, RPS)],
                        out_hbm.at[c, pl.ds(s * RPS, RPS)])

    mesh1 = plsc.VectorSubcoreMesh(core_axis_name="c", subcore_axis_name="s",
                                   num_cores=1, num_subcores=NS)
    IB = 4
    RB = 2

    @functools.partial(
        pl.kernel,
        out_type=jax.ShapeDtypeStruct((NPAD, H), jnp.float32),
        mesh=mesh1,
        scratch_types=[
            pltpu.VMEM_SHARED((NPAD, H), jnp.float32),
            [pltpu.VMEM((2, CH), jnp.int32) for _ in range(IB)],
            [pltpu.VMEM((CH, H), jnp.float32) for _ in range(RB)],
            [pltpu.SemaphoreType.DMA for _ in range(IB)],
            [pltpu.SemaphoreType.DMA for _ in range(RB)],
            [pltpu.SemaphoreType.DMA for _ in range(RB)],
        ],
    )
    def sc_aggregate(sd_hbm, h_hbm, zeros2_hbm, out_hbm,
                     agg_sh, ibuf, rows, isem, gsem, ssem):
        s = lax.axis_index("s")
        kw = KAG
        start = s * KAG
        pltpu.sync_copy(zeros2_hbm.at[pl.ds(s * RPS, RPS)],
                        agg_sh.at[pl.ds(s * RPS, RPS)])
        plsc.subcore_barrier()

        def icopy(j, q):
            return pltpu.make_async_copy(sd_hbm.at[start + j], ibuf[q], isem[q])

        def gather(q, b):
            return pltpu.make_async_copy(h_hbm.at[ibuf[q].at[0]], rows[b], gsem[b])

        def scatter(q, b):
            return pltpu.make_async_copy(rows[b], agg_sh.at[ibuf[q].at[1]], ssem[b])

        for q in range(3):
            icopy(q, q).start()
        icopy(0, 0).wait()
        gather(0, 0).start()

        def body(g, carry):
            for b4 in range(4):
                j = g * 4 + b4
                q, b = b4, b4 % 2
                qn, bn = (b4 + 1) % 4, (b4 + 1) % 2

                @pl.when(j >= 1)
                def _():
                    scatter((b4 + 3) % 4, bn).wait()

                @pl.when(j + 3 < kw)
                def _():
                    icopy(j + 3, (b4 + 3) % 4).start()

                @pl.when(j + 1 < kw)
                def _():
                    icopy(j + 1, qn).wait()
                    gather(qn, bn).start()

                gather(q, b).wait()
                scatter(q, b).start(add=True)
            return carry

        lax.fori_loop(0, kw // 4, body, 0)
        scatter(3, 1).wait()
        plsc.subcore_barrier()
        pltpu.sync_copy(agg_sh.at[pl.ds(s * RPS, RPS)],
                        out_hbm.at[pl.ds(s * RPS, RPS)])

    return sc_degree, sc_aggregate


BLK = 400
NBLK = N // BLK

_full = lambda shape: pl.BlockSpec(shape, lambda i: (0,) * len(shape))


def _mlp1_body(d0, d1, emb, w1, b1, w2, b2, out):
    v1 = jnp.dot(emb[...], w1[...], preferred_element_type=jnp.float32)
    scale = 1.0 + d0[...] + d1[...]
    z = jnp.maximum(scale * v1 + b1[...], 0.0)
    out[...] = jnp.dot(z, w2[...], preferred_element_type=jnp.float32) + b2[...]


def _tc_layer1(d0, d1, emb, w1, b1, w2, b2):
    return pl.pallas_call(
        _mlp1_body,
        grid=(NBLK,),
        in_specs=[
            pl.BlockSpec((BLK, 1), lambda i: (i, 0)),
            pl.BlockSpec((BLK, 1), lambda i: (i, 0)),
            _full((1, H)), _full((H, H)), _full((1, H)), _full((H, H)), _full((1, H)),
        ],
        out_specs=pl.BlockSpec((BLK, H), lambda i: (i, 0)),
        out_shape=jax.ShapeDtypeStruct((N, H), jnp.float32),
    )(d0, d1, emb, w1, b1, w2, b2)


def _mlp_body(h, a, w1, b1, w2, b2, out):
    z = h[...] + a[...]
    z = jnp.maximum(jnp.dot(z, w1[...], preferred_element_type=jnp.float32) + b1[...], 0.0)
    out[...] = jnp.dot(z, w2[...], preferred_element_type=jnp.float32) + b2[...]


def _tc_layer(h, a, w1, b1, w2, b2):
    rb = pl.BlockSpec((BLK, H), lambda i: (i, 0))
    return pl.pallas_call(
        _mlp_body,
        grid=(NBLK,),
        in_specs=[rb, rb, _full((H, H)), _full((1, H)), _full((H, H)), _full((1, H))],
        out_specs=rb,
        out_shape=jax.ShapeDtypeStruct((N, H), jnp.float32),
    )(h, a, w1, b1, w2, b2)


def _final_body(h, a, w1, b1, w2, b2, bat, wp, bp, out, pooled, counts):
    i = pl.program_id(0)

    @pl.when(i == 0)
    def _init():
        pooled[...] = jnp.zeros_like(pooled)
        counts[...] = jnp.zeros_like(counts)

    z = h[...] + a[...]
    z = jnp.maximum(jnp.dot(z, w1[...], preferred_element_type=jnp.float32) + b1[...], 0.0)
    h3 = jnp.dot(z, w2[...], preferred_element_type=jnp.float32) + b2[...]
    gids = lax.broadcasted_iota(jnp.int32, (1, G), 1)
    oh = (bat[...] == gids).astype(jnp.float32)
    pooled[...] += lax.dot_general(oh, h3, (((0,), (0,)), ((), ())),
                                   preferred_element_type=jnp.float32)
    counts[...] += lax.dot_general(oh, jnp.ones((BLK, 1), jnp.float32),
                                   (((0,), (0,)), ((), ())),
                                   preferred_element_type=jnp.float32)

    @pl.when(i == NBLK - 1)
    def _head():
        pm = pooled[...] / jnp.maximum(counts[...], 1.0)
        out[...] = jnp.dot(pm, wp[...], preferred_element_type=jnp.float32) + bp[...]


def _tc_final(h, a, w1, b1, w2, b2, bat, wp, bp):
    rb = pl.BlockSpec((BLK, H), lambda i: (i, 0))
    return pl.pallas_call(
        _final_body,
        grid=(NBLK,),
        in_specs=[rb, rb, _full((H, H)), _full((1, H)), _full((H, H)), _full((1, H)),
                  pl.BlockSpec((BLK, 1), lambda i: (i, 0)), _full((H, H)), _full((1, H))],
        out_specs=_full((G, H)),
        out_shape=jax.ShapeDtypeStruct((G, H), jnp.float32),
        scratch_shapes=[pltpu.VMEM((G, H), jnp.float32), pltpu.VMEM((G, 1), jnp.float32)],
    )(h, a, w1, b1, w2, b2, bat, wp, bp)


def kernel(x, edge_index, edge_attr, batch, emb,
           W1_0, b1_0, W2_0, b2_0, W1_1, b1_1, W2_1, b2_1,
           W1_2, b1_2, W2_2, b2_2, Wp, bp):
    src = edge_index[0].astype(jnp.int32)
    dst = edge_index[1].astype(jnp.int32)
    pad = EPAD - E
    src_p = jnp.concatenate([src, jnp.zeros((pad,), jnp.int32)])
    dst_p = jnp.concatenate([dst, jnp.full((pad,), N, jnp.int32)])
    dst3 = dst_p.reshape(NW, KCH, CH)
    sd = jnp.stack([src_p.reshape(TOT, CH), dst_p.reshape(TOT, CH)], axis=1)
    zeros1 = jnp.zeros((NPAD,), jnp.float32)
    zeros2 = jnp.zeros((NPAD, H), jnp.float32)

    sc_degree, sc_aggregate = _sc_kernels()
    deg = sc_degree(dst3, zeros1)
    d0 = deg[0, :N].reshape(N, 1)
    d1 = deg[1, :N].reshape(N, 1)

    emb2 = emb.reshape(1, H)
    b1r = [b1_0.reshape(1, H), b1_1.reshape(1, H), b1_2.reshape(1, H)]
    b2r = [b2_0.reshape(1, H), b2_1.reshape(1, H), b2_2.reshape(1, H)]

    h1 = _tc_layer1(d0, d1, emb2, W1_0, b1r[0], W2_0, b2r[0])

    agg1 = sc_aggregate(sd, h1, zeros2)
    h2 = _tc_layer(h1, agg1, W1_1, b1r[1], W2_1, b2r[1])

    agg2 = sc_aggregate(sd, h2, zeros2)
    bat2 = batch.astype(jnp.int32).reshape(N, 1)
    Wp_pad = jnp.zeros((H, H), jnp.float32).at[:, :C].set(Wp)
    bp_pad = jnp.zeros((1, H), jnp.float32).at[0, :C].set(bp)
    outp = _tc_final(h2, agg2, W1_2, b1r[2], W2_2, b2r[2],
                     bat2, Wp_pad, bp_pad)
    return outp[:, :C]


# speedup vs baseline: 1.0309x; 1.0309x over previous
import functools

import jax
import jax.numpy as jnp
from jax import lax
from jax.experimental import pallas as pl
from jax.experimental.pallas import tpu as pltpu
from jax.experimental.pallas import tpu_sc as plsc

N = 10000
E = 320000
H = 128
G = 128
C = 10

NC = 2
NS = 16
NW = NC * NS
CH = 128
KCH = 80
PTE = KCH * CH
EPAD = NW * PTE
TOT = EPAD // CH
KAG = TOT // NS
NPAD = 10240
RPS = NPAD // NS

@functools.cache
def _sc_kernels():
    mesh = plsc.VectorSubcoreMesh(core_axis_name="c", subcore_axis_name="s",
                                  num_cores=NC, num_subcores=NS)

    @functools.partial(
        pl.kernel,
        out_type=jax.ShapeDtypeStruct((NC, NPAD), jnp.float32),
        mesh=mesh,
        scratch_types=[
            pltpu.VMEM_SHARED((NPAD,), jnp.float32),
            pltpu.VMEM((KCH, CH), jnp.int32),
            pltpu.VMEM((CH,), jnp.float32),
        ],
    )
    def sc_degree(dst3_hbm, zeros1_hbm, out_hbm, deg_sh, idst_v, ones_v):
        c = lax.axis_index("c")
        s = lax.axis_index("s")
        w = s * NC + c
        pltpu.sync_copy(zeros1_hbm.at[pl.ds(s * RPS, RPS)],
                        deg_sh.at[pl.ds(s * RPS, RPS)])
        for i in range(CH // 16):
            ones_v[pl.ds(i * 16, 16)] = jnp.full((16,), 1.0, jnp.float32)
        plsc.subcore_barrier()
        pltpu.sync_copy(dst3_hbm.at[w], idst_v)

        def body(j, carry):
            pltpu.sync_copy(ones_v, deg_sh.at[idst_v.at[j]], add=True)
            return carry

        lax.fori_loop(0, KCH, body, 0)
        plsc.subcore_barrier()
        pltpu.sync_copy(deg_sh.at[pl.ds(s * RPS, RPS)],
                        out_hbm.at[c, pl.ds(s * RPS, RPS)])

    IB = 4
    RB = 2

    @functools.partial(
        pl.kernel,
        out_type=jax.ShapeDtypeStruct((NPAD, H), jnp.float32),
        mesh=mesh,
        scratch_types=[
            pltpu.VMEM_SHARED((NPAD, H), jnp.float32),
            [pltpu.VMEM((2, CH), jnp.int32) for _ in range(IB)],
            [pltpu.VMEM((CH, H), jnp.float32) for _ in range(RB)],
            [pltpu.SemaphoreType.DMA for _ in range(IB)],
            [pltpu.SemaphoreType.DMA for _ in range(RB)],
            [pltpu.SemaphoreType.DMA for _ in range(RB)],
        ],
    )
    def sc_aggregate(sd_hbm, h_hbm, zeros2_hbm, out_hbm,
                     agg_sh, ibuf, rows, isem, gsem, ssem):
        c = lax.axis_index("c")
        s = lax.axis_index("s")
        kw = KAG
        start = s * KAG

        def icopy(j, q):
            return pltpu.make_async_copy(sd_hbm.at[start + j], ibuf[q], isem[q])

        def gather(q, b):
            return pltpu.make_async_copy(h_hbm.at[ibuf[q].at[0]], rows[b], gsem[b])

        def scatter(q, b):
            return pltpu.make_async_copy(rows[b], agg_sh.at[ibuf[q].at[1]], ssem[b])

        @pl.when(c == 0)
        def _core0():
            pltpu.sync_copy(zeros2_hbm.at[pl.ds(s * RPS, RPS)],
                            agg_sh.at[pl.ds(s * RPS, RPS)])

        plsc.subcore_barrier()

        @pl.when(c == 0)
        def _core0_main():
            for q in range(3):
                icopy(q, q).start()
            icopy(0, 0).wait()
            gather(0, 0).start()

            def body(g, carry):
                for b4 in range(4):
                    j = g * 4 + b4
                    q, b = b4, b4 % 2
                    qn, bn = (b4 + 1) % 4, (b4 + 1) % 2

                    @pl.when(j >= 1)
                    def _():
                        scatter((b4 + 3) % 4, bn).wait()

                    @pl.when(j + 3 < kw)
                    def _():
                        icopy(j + 3, (b4 + 3) % 4).start()

                    @pl.when(j + 1 < kw)
                    def _():
                        icopy(j + 1, qn).wait()
                        gather(qn, bn).start()

                    gather(q, b).wait()
                    scatter(q, b).start(add=True)
                return carry

            lax.fori_loop(0, kw // 4, body, 0)
            scatter(3, 1).wait()

        plsc.subcore_barrier()

        @pl.when(c == 0)
        def _core0_out():
            pltpu.sync_copy(agg_sh.at[pl.ds(s * RPS, RPS)],
                            out_hbm.at[pl.ds(s * RPS, RPS)])

    return sc_degree, sc_aggregate


BLK = 400
NBLK = N // BLK

_full = lambda shape: pl.BlockSpec(shape, lambda i: (0,) * len(shape))


def _mlp1_body(d0, d1, emb, w1, b1, w2, b2, out):
    v1 = jnp.dot(emb[...], w1[...], preferred_element_type=jnp.float32)
    scale = 1.0 + d0[...] + d1[...]
    z = jnp.maximum(scale * v1 + b1[...], 0.0)
    out[...] = jnp.dot(z, w2[...], preferred_element_type=jnp.float32) + b2[...]


def _tc_layer1(d0, d1, emb, w1, b1, w2, b2):
    return pl.pallas_call(
        _mlp1_body,
        grid=(NBLK,),
        in_specs=[
            pl.BlockSpec((BLK, 1), lambda i: (i, 0)),
            pl.BlockSpec((BLK, 1), lambda i: (i, 0)),
            _full((1, H)), _full((H, H)), _full((1, H)), _full((H, H)), _full((1, H)),
        ],
        out_specs=pl.BlockSpec((BLK, H), lambda i: (i, 0)),
        out_shape=jax.ShapeDtypeStruct((N, H), jnp.float32),
    )(d0, d1, emb, w1, b1, w2, b2)


def _mlp_body(h, a, w1, b1, w2, b2, out):
    z = h[...] + a[...]
    z = jnp.maximum(jnp.dot(z, w1[...], preferred_element_type=jnp.float32) + b1[...], 0.0)
    out[...] = jnp.dot(z, w2[...], preferred_element_type=jnp.float32) + b2[...]


def _tc_layer(h, a, w1, b1, w2, b2):
    rb = pl.BlockSpec((BLK, H), lambda i: (i, 0))
    return pl.pallas_call(
        _mlp_body,
        grid=(NBLK,),
        in_specs=[rb, rb, _full((H, H)), _full((1, H)), _full((H, H)), _full((1, H))],
        out_specs=rb,
        out_shape=jax.ShapeDtypeStruct((N, H), jnp.float32),
    )(h, a, w1, b1, w2, b2)


def _final_body(h, a, w1, b1, w2, b2, bat, wp, bp, out, pooled, counts):
    i = pl.program_id(0)

    @pl.when(i == 0)
    def _init():
        pooled[...] = jnp.zeros_like(pooled)
        counts[...] = jnp.zeros_like(counts)

    z = h[...] + a[...]
    z = jnp.maximum(jnp.dot(z, w1[...], preferred_element_type=jnp.float32) + b1[...], 0.0)
    h3 = jnp.dot(z, w2[...], preferred_element_type=jnp.float32) + b2[...]
    gids = lax.broadcasted_iota(jnp.int32, (1, G), 1)
    oh = (bat[...] == gids).astype(jnp.float32)
    pooled[...] += lax.dot_general(oh, h3, (((0,), (0,)), ((), ())),
                                   preferred_element_type=jnp.float32)
    counts[...] += lax.dot_general(oh, jnp.ones((BLK, 1), jnp.float32),
                                   (((0,), (0,)), ((), ())),
                                   preferred_element_type=jnp.float32)

    @pl.when(i == NBLK - 1)
    def _head():
        pm = pooled[...] / jnp.maximum(counts[...], 1.0)
        out[...] = jnp.dot(pm, wp[...], preferred_element_type=jnp.float32) + bp[...]


def _tc_final(h, a, w1, b1, w2, b2, bat, wp, bp):
    rb = pl.BlockSpec((BLK, H), lambda i: (i, 0))
    return pl.pallas_call(
        _final_body,
        grid=(NBLK,),
        in_specs=[rb, rb, _full((H, H)), _full((1, H)), _full((H, H)), _full((1, H)),
                  pl.BlockSpec((BLK, 1), lambda i: (i, 0)), _full((H, H)), _full((1, H))],
        out_specs=_full((G, H)),
        out_shape=jax.ShapeDtypeStruct((G, H), jnp.float32),
        scratch_shapes=[pltpu.VMEM((G, H), jnp.float32), pltpu.VMEM((G, 1), jnp.float32)],
    )(h, a, w1, b1, w2, b2, bat, wp, bp)


def kernel(x, edge_index, edge_attr, batch, emb,
           W1_0, b1_0, W2_0, b2_0, W1_1, b1_1, W2_1, b2_1,
           W1_2, b1_2, W2_2, b2_2, Wp, bp):
    src = edge_index[0].astype(jnp.int32)
    dst = edge_index[1].astype(jnp.int32)
    pad = EPAD - E
    src_p = jnp.concatenate([src, jnp.zeros((pad,), jnp.int32)])
    dst_p = jnp.concatenate([dst, jnp.full((pad,), N, jnp.int32)])
    dst3 = dst_p.reshape(NW, KCH, CH)
    sd = jnp.stack([src_p.reshape(TOT, CH), dst_p.reshape(TOT, CH)], axis=1)
    zeros1 = jnp.zeros((NPAD,), jnp.float32)
    zeros2 = jnp.zeros((NPAD, H), jnp.float32)

    sc_degree, sc_aggregate = _sc_kernels()
    deg = sc_degree(dst3, zeros1)
    d0 = deg[0, :N].reshape(N, 1)
    d1 = deg[1, :N].reshape(N, 1)

    emb2 = emb.reshape(1, H)
    b1r = [b1_0.reshape(1, H), b1_1.reshape(1, H), b1_2.reshape(1, H)]
    b2r = [b2_0.reshape(1, H), b2_1.reshape(1, H), b2_2.reshape(1, H)]

    h1 = _tc_layer1(d0, d1, emb2, W1_0, b1r[0], W2_0, b2r[0])

    agg1 = sc_aggregate(sd, h1, zeros2)
    h2 = _tc_layer(h1, agg1, W1_1, b1r[1], W2_1, b2r[1])

    agg2 = sc_aggregate(sd, h2, zeros2)
    bat2 = batch.astype(jnp.int32).reshape(N, 1)
    Wp_pad = jnp.zeros((H, H), jnp.float32).at[:, :C].set(Wp)
    bp_pad = jnp.zeros((1, H), jnp.float32).at[0, :C].set(bp)
    outp = _tc_final(h2, agg2, W1_2, b1r[2], W2_2, b2r[2],
                     bat2, Wp_pad, bp_pad)
    return outp[:, :C]


# speedup vs baseline: 1.3780x; 1.3368x over previous
import functools

import jax
import jax.numpy as jnp
from jax import lax
from jax.experimental import pallas as pl
from jax.experimental.pallas import tpu as pltpu
from jax.experimental.pallas import tpu_sc as plsc

N = 10000
E = 320000
H = 128
G = 128
C = 10

NC = 2
NS = 16
NW = NC * NS
CH = 128
KCH = 80
PTE = KCH * CH
EPAD = NW * PTE
TOT = EPAD // CH
K0 = 140
K1 = 20
NPAD = 10240
RPS = NPAD // NS

@functools.cache
def _sc_kernels():
    mesh = plsc.VectorSubcoreMesh(core_axis_name="c", subcore_axis_name="s",
                                  num_cores=NC, num_subcores=NS)

    @functools.partial(
        pl.kernel,
        out_type=jax.ShapeDtypeStruct((NC, NPAD), jnp.float32),
        mesh=mesh,
        scratch_types=[
            pltpu.VMEM_SHARED((NPAD,), jnp.float32),
            pltpu.VMEM((KCH, CH), jnp.int32),
            pltpu.VMEM((CH,), jnp.float32),
        ],
    )
    def sc_degree(dst3_hbm, zeros1_hbm, out_hbm, deg_sh, idst_v, ones_v):
        c = lax.axis_index("c")
        s = lax.axis_index("s")
        w = s * NC + c
        pltpu.sync_copy(zeros1_hbm.at[pl.ds(s * RPS, RPS)],
                        deg_sh.at[pl.ds(s * RPS, RPS)])
        for i in range(CH // 16):
            ones_v[pl.ds(i * 16, 16)] = jnp.full((16,), 1.0, jnp.float32)
        plsc.subcore_barrier()
        pltpu.sync_copy(dst3_hbm.at[w], idst_v)

        def body(j, carry):
            pltpu.sync_copy(ones_v, deg_sh.at[idst_v.at[j]], add=True)
            return carry

        lax.fori_loop(0, KCH, body, 0)
        plsc.subcore_barrier()
        pltpu.sync_copy(deg_sh.at[pl.ds(s * RPS, RPS)],
                        out_hbm.at[c, pl.ds(s * RPS, RPS)])

    IB = 4
    RB = 2

    @functools.partial(
        pl.kernel,
        out_type=jax.ShapeDtypeStruct((NC, NPAD, H), jnp.float32),
        mesh=mesh,
        scratch_types=[
            pltpu.VMEM_SHARED((NPAD, H), jnp.float32),
            [pltpu.VMEM((2, CH), jnp.int32) for _ in range(IB)],
            [pltpu.VMEM((CH, H), jnp.float32) for _ in range(RB)],
            [pltpu.SemaphoreType.DMA for _ in range(IB)],
            [pltpu.SemaphoreType.DMA for _ in range(RB)],
            [pltpu.SemaphoreType.DMA for _ in range(RB)],
        ],
    )
    def sc_aggregate(sd_hbm, h_hbm, out_hbm,
                     agg_sh, ibuf, rows, isem, gsem, ssem):
        c = lax.axis_index("c")
        s = lax.axis_index("s")
        kw = jnp.where(c == 0, K0, K1)
        start = jnp.where(c == 0, s * K0, 16 * K0 + s * K1)

        def zbody(k, carry):
            i = lax.shift_right_logical(k, 3)
            l = lax.bitwise_and(k, 7)
            rows[0][i, pl.ds(l * 16, 16)] = jnp.zeros((16,), jnp.float32)
            return carry

        lax.fori_loop(0, CH * 8, zbody, 0)
        for r in range(RPS // CH):
            pltpu.sync_copy(rows[0], agg_sh.at[pl.ds(s * RPS + r * CH, CH)])
        plsc.subcore_barrier()

        def icopy(j, q):
            return pltpu.make_async_copy(sd_hbm.at[start + j], ibuf[q], isem[q])

        def gather(q, b):
            return pltpu.make_async_copy(h_hbm.at[ibuf[q].at[0]], rows[b], gsem[b])

        def scatter(q, b):
            return pltpu.make_async_copy(rows[b], agg_sh.at[ibuf[q].at[1]], ssem[b])

        for q in range(3):
            icopy(q, q).start()
        icopy(0, 0).wait()
        gather(0, 0).start()

        def body(g, carry):
            for b4 in range(4):
                j = g * 4 + b4
                q, b = b4, b4 % 2
                qn, bn = (b4 + 1) % 4, (b4 + 1) % 2

                @pl.when(j >= 1)
                def _():
                    scatter((b4 + 3) % 4, bn).wait()

                @pl.when(j + 3 < kw)
                def _():
                    icopy(j + 3, (b4 + 3) % 4).start()

                @pl.when(j + 1 < kw)
                def _():
                    icopy(j + 1, qn).wait()
                    gather(qn, bn).start()

                gather(q, b).wait()
                scatter(q, b).start(add=True)
            return carry

        lax.fori_loop(0, kw // 4, body, 0)
        scatter(3, 1).wait()
        plsc.subcore_barrier()
        pltpu.sync_copy(agg_sh.at[pl.ds(s * RPS, RPS)],
                        out_hbm.at[c, pl.ds(s * RPS, RPS)])

    return sc_degree, sc_aggregate


BLK = 400
NBLK = N // BLK

_full = lambda shape: pl.BlockSpec(shape, lambda i: (0,) * len(shape))


def _mlp1_body(d0, d1, emb, w1, b1, w2, b2, out):
    v1 = jnp.dot(emb[...], w1[...], preferred_element_type=jnp.float32)
    scale = 1.0 + d0[...] + d1[...]
    z = jnp.maximum(scale * v1 + b1[...], 0.0)
    out[...] = jnp.dot(z, w2[...], preferred_element_type=jnp.float32) + b2[...]


def _tc_layer1(d0, d1, emb, w1, b1, w2, b2):
    return pl.pallas_call(
        _mlp1_body,
        grid=(NBLK,),
        in_specs=[
            pl.BlockSpec((BLK, 1), lambda i: (i, 0)),
            pl.BlockSpec((BLK, 1), lambda i: (i, 0)),
            _full((1, H)), _full((H, H)), _full((1, H)), _full((H, H)), _full((1, H)),
        ],
        out_specs=pl.BlockSpec((BLK, H), lambda i: (i, 0)),
        out_shape=jax.ShapeDtypeStruct((N, H), jnp.float32),
    )(d0, d1, emb, w1, b1, w2, b2)


def _mlp_body(h, a0, a1, w1, b1, w2, b2, out):
    z = h[...] + a0[...] + a1[...]
    z = jnp.maximum(jnp.dot(z, w1[...], preferred_element_type=jnp.float32) + b1[...], 0.0)
    out[...] = jnp.dot(z, w2[...], preferred_element_type=jnp.float32) + b2[...]


def _tc_layer(h, a0, a1, w1, b1, w2, b2):
    rb = pl.BlockSpec((BLK, H), lambda i: (i, 0))
    return pl.pallas_call(
        _mlp_body,
        grid=(NBLK,),
        in_specs=[rb, rb, rb, _full((H, H)), _full((1, H)), _full((H, H)), _full((1, H))],
        out_specs=rb,
        out_shape=jax.ShapeDtypeStruct((N, H), jnp.float32),
    )(h, a0, a1, w1, b1, w2, b2)


def _final_body(h, a0, a1, w1, b1, w2, b2, bat, wp, bp, out, pooled, counts):
    i = pl.program_id(0)

    @pl.when(i == 0)
    def _init():
        pooled[...] = jnp.zeros_like(pooled)
        counts[...] = jnp.zeros_like(counts)

    z = h[...] + a0[...] + a1[...]
    z = jnp.maximum(jnp.dot(z, w1[...], preferred_element_type=jnp.float32) + b1[...], 0.0)
    h3 = jnp.dot(z, w2[...], preferred_element_type=jnp.float32) + b2[...]
    gids = lax.broadcasted_iota(jnp.int32, (1, G), 1)
    oh = (bat[...] == gids).astype(jnp.float32)
    pooled[...] += lax.dot_general(oh, h3, (((0,), (0,)), ((), ())),
                                   preferred_element_type=jnp.float32)
    counts[...] += lax.dot_general(oh, jnp.ones((BLK, 1), jnp.float32),
                                   (((0,), (0,)), ((), ())),
                                   preferred_element_type=jnp.float32)

    @pl.when(i == NBLK - 1)
    def _head():
        pm = pooled[...] / jnp.maximum(counts[...], 1.0)
        out[...] = jnp.dot(pm, wp[...], preferred_element_type=jnp.float32) + bp[...]


def _tc_final(h, a0, a1, w1, b1, w2, b2, bat, wp, bp):
    rb = pl.BlockSpec((BLK, H), lambda i: (i, 0))
    return pl.pallas_call(
        _final_body,
        grid=(NBLK,),
        in_specs=[rb, rb, rb, _full((H, H)), _full((1, H)), _full((H, H)), _full((1, H)),
                  pl.BlockSpec((BLK, 1), lambda i: (i, 0)), _full((H, H)), _full((1, H))],
        out_specs=_full((G, H)),
        out_shape=jax.ShapeDtypeStruct((G, H), jnp.float32),
        scratch_shapes=[pltpu.VMEM((G, H), jnp.float32), pltpu.VMEM((G, 1), jnp.float32)],
    )(h, a0, a1, w1, b1, w2, b2, bat, wp, bp)


def kernel(x, edge_index, edge_attr, batch, emb,
           W1_0, b1_0, W2_0, b2_0, W1_1, b1_1, W2_1, b2_1,
           W1_2, b1_2, W2_2, b2_2, Wp, bp):
    src = edge_index[0].astype(jnp.int32)
    dst = edge_index[1].astype(jnp.int32)
    pad = EPAD - E
    src_p = jnp.concatenate([src, jnp.zeros((pad,), jnp.int32)])
    dst_p = jnp.concatenate([dst, jnp.full((pad,), N, jnp.int32)])
    dst3 = dst_p.reshape(NW, KCH, CH)
    sd = jnp.stack([src_p.reshape(TOT, CH), dst_p.reshape(TOT, CH)], axis=1)
    zeros1 = jnp.zeros((NPAD,), jnp.float32)

    sc_degree, sc_aggregate = _sc_kernels()
    deg = sc_degree(dst3, zeros1)
    d0 = deg[0, :N].reshape(N, 1)
    d1 = deg[1, :N].reshape(N, 1)

    emb2 = emb.reshape(1, H)
    b1r = [b1_0.reshape(1, H), b1_1.reshape(1, H), b1_2.reshape(1, H)]
    b2r = [b2_0.reshape(1, H), b2_1.reshape(1, H), b2_2.reshape(1, H)]

    h1 = _tc_layer1(d0, d1, emb2, W1_0, b1r[0], W2_0, b2r[0])

    agg1 = sc_aggregate(sd, h1)
    h2 = _tc_layer(h1, agg1[0], agg1[1], W1_1, b1r[1], W2_1, b2r[1])

    agg2 = sc_aggregate(sd, h2)
    bat2 = batch.astype(jnp.int32).reshape(N, 1)
    Wp_pad = jnp.zeros((H, H), jnp.float32).at[:, :C].set(Wp)
    bp_pad = jnp.zeros((1, H), jnp.float32).at[0, :C].set(bp)
    outp = _tc_final(h2, agg2[0], agg2[1], W1_2, b1r[2], W2_2, b2r[2],
                     bat2, Wp_pad, bp_pad)
    return outp[:, :C]
